# baseline (speedup 1.0000x reference)
import jax
import jax.numpy as jnp
import numpy as np
from jax.experimental import pallas as pl
from jax.experimental.pallas import tpu as pltpu

BT = 128
W1 = 30
N1 = 32 * W1
A1 = 28 * W1
J1 = 24
K1 = 88
W2 = 16
N2 = 17 * W2
A2 = 14 * W2
J2 = 16
K2 = 56
F1 = 560


def _fused_kernel(x_ref, l1_ref, b1_ref, l2_ref, b2_ref,
                  fc1w_ref, fc1b_ref, fc2w_ref, fc2b_ref,
                  o_ref, xps, a1s, p2s, a2s, p3s):
    f32 = jnp.float32
    bf16 = jnp.bfloat16

    xps[...] = jnp.zeros(xps.shape, f32)
    xv = jnp.transpose(x_ref[...], (1, 0))
    for rr in range(28):
        xps[pl.ds((rr + 1) * W1 + 1, 28), :] = xv[rr * 28:(rr + 1) * 28, :]

    for c in range(35):
        base = c * J1
        r = xps[pl.ds(base, K1), :]
        res = jnp.dot(l1_ref[...], r, preferred_element_type=f32)
        res = jnp.maximum(res + b1_ref[...], 0.0)
        for co in range(10):
            a1s[co, pl.ds(base, J1), :] = res[co * J1:(co + 1) * J1, :]

    p2s[...] = jnp.zeros(p2s.shape, f32)
    for ci in range(10):
        for ph in range(14):
            m = None
            for di in range(2):
                for dj in range(2):
                    v = a1s[ci, pl.ds((2 * ph + di) * W1 + dj, 14, stride=2), :]
                    m = v if m is None else jnp.maximum(m, v)
            p2s[ci, pl.ds((ph + 1) * W2 + 1, 14), :] = m

    for c in range(14):
        base = c * J2
        r = jnp.concatenate(
            [p2s[ci, pl.ds(base, K2), :] for ci in range(10)], axis=0)
        res = jnp.dot(l2_ref[...], r, preferred_element_type=f32)
        res = jnp.maximum(res + b2_ref[...], 0.0)
        for co in range(10):
            a2s[co, pl.ds(base, J2), :] = res[co * J2:(co + 1) * J2, :]

    for ci in range(10):
        for ph in range(7):
            m = None
            for di in range(2):
                for dj in range(2):
                    v = a2s[ci, pl.ds((2 * ph + di) * W2 + dj, 8, stride=2), :]
                    m = v if m is None else jnp.maximum(m, v)
            p3s[pl.ds(ci * 56 + ph * 8, 8), :] = m

    h1 = jnp.dot(fc1w_ref[...], p3s[...], preferred_element_type=f32)
    h1 = jnp.maximum(h1 + fc1b_ref[...], 0.0)
    out = jnp.dot(fc2w_ref[...], h1, preferred_element_type=f32)
    o_ref[...] = out + fc2b_ref[...]


@jax.jit
def kernel(x, w1, b1, w2, b2, fc1_w, fc1_b, fc2_w, fc2_b):
    B = x.shape[0]
    f32 = jnp.float32
    bf16 = jnp.bfloat16

    xt = x[:, 0].reshape(B, 784)

    oh1 = np.zeros((9, J1, K1), np.float32)
    oh2 = np.zeros((9, J2, K2), np.float32)
    for t in range(9):
        for p in range(J1):
            oh1[t, p, p + (t // 3) * W1 + (t % 3)] = 1.0
        for p in range(J2):
            oh2[t, p, p + (t // 3) * W2 + (t % 3)] = 1.0
    w1s = w1[:, 0, :10].T
    l1m = jnp.einsum('ct,tpk->cpk', w1s,
                     jnp.asarray(oh1)).reshape(240, K1)
    b1r = jnp.broadcast_to(jnp.repeat(b1[0, :10], J1)[:, None], (240, BT))

    w2s = jnp.transpose(w2[:, :10, :10], (2, 1, 0))
    l2m = jnp.einsum('cit,tpk->cpik', w2s,
                     jnp.asarray(oh2)).reshape(160, 10 * K2)
    b2r = jnp.broadcast_to(jnp.repeat(b2[0, :10], J2)[:, None], (160, BT))

    fw = fc1_w[:, :10, :].reshape(7, 7, 10, 32)
    fwp = jnp.zeros((10, 7, 8, 32), f32)
    fwp = fwp.at[:, :, :7, :].set(jnp.transpose(fw, (2, 0, 1, 3)))
    fc1wm = fwp.reshape(F1, 32).T
    fc1bb = jnp.broadcast_to(fc1_b.reshape(32, 1), (32, BT))

    fc2wm = jnp.zeros((16, 32), f32).at[:10].set(fc2_w[:, :10].T)
    fc2bb = jnp.zeros((16, BT), f32).at[:10].set(
        jnp.broadcast_to(fc2_b[0, :10, None], (10, BT)))

    grid = (B // BT,)
    out = pl.pallas_call(
        _fused_kernel,
        out_shape=jax.ShapeDtypeStruct((16, B), f32),
        grid=grid,
        in_specs=[
            pl.BlockSpec((BT, 784), lambda b: (b, 0)),
            pl.BlockSpec((240, K1), lambda b: (0, 0)),
            pl.BlockSpec((240, BT), lambda b: (0, 0)),
            pl.BlockSpec((160, 10 * K2), lambda b: (0, 0)),
            pl.BlockSpec((160, BT), lambda b: (0, 0)),
            pl.BlockSpec((32, F1), lambda b: (0, 0)),
            pl.BlockSpec((32, BT), lambda b: (0, 0)),
            pl.BlockSpec((16, 32), lambda b: (0, 0)),
            pl.BlockSpec((16, BT), lambda b: (0, 0)),
        ],
        out_specs=pl.BlockSpec((16, BT), lambda b: (0, b)),
        scratch_shapes=[
            pltpu.VMEM((N1, BT), f32),
            pltpu.VMEM((10, A1, BT), f32),
            pltpu.VMEM((10, N2, BT), f32),
            pltpu.VMEM((10, A2, BT), f32),
            pltpu.VMEM((F1, BT), f32),
        ],
        compiler_params=pltpu.CompilerParams(
            dimension_semantics=("parallel",)),
    )(xt, l1m, b1r, l2m, b2r, fc1wm, fc1bb, fc2wm, fc2bb)

    return out[:10].T


# speedup vs baseline: 1.2469x; 1.2469x over previous
import jax
import jax.numpy as jnp
import numpy as np
from jax.experimental import pallas as pl
from jax.experimental.pallas import tpu as pltpu

BT = 128
W1 = 30
N1 = 32 * W1
A1 = 28 * W1
J1 = 24
K1 = 88
W2 = 16
N2 = 17 * W2
A2 = 14 * W2
J2 = 16
K2 = 56
F1 = 560


def _fused_kernel(x_ref, l1_ref, b1_ref, l2_ref, b2_ref,
                  fc1w_ref, fc1b_ref, fc2w_ref, fc2b_ref,
                  o_ref, a1s, p2s, a2s, p3s):
    f32 = jnp.float32
    bf16 = jnp.bfloat16

    for c in range(35):
        base = c * J1
        r = x_ref[pl.ds(base, K1), :]
        res = jnp.dot(l1_ref[...], r, preferred_element_type=f32)
        res = jnp.maximum(res + b1_ref[...], 0.0)
        for co in range(10):
            a1s[co, pl.ds(base, J1), :] = res[co * J1:(co + 1) * J1, :]

    p2s[...] = jnp.zeros(p2s.shape, f32)
    for ci in range(10):
        for ph in range(14):
            m = None
            for di in range(2):
                for dj in range(2):
                    v = a1s[ci, pl.ds((2 * ph + di) * W1 + dj, 14, stride=2), :]
                    m = v if m is None else jnp.maximum(m, v)
            p2s[ci, pl.ds((ph + 1) * W2 + 1, 14), :] = m

    for c in range(14):
        base = c * J2
        r = jnp.concatenate(
            [p2s[ci, pl.ds(base, K2), :] for ci in range(10)], axis=0)
        res = jnp.dot(l2_ref[...], r, preferred_element_type=f32)
        res = jnp.maximum(res + b2_ref[...], 0.0)
        for co in range(10):
            a2s[co, pl.ds(base, J2), :] = res[co * J2:(co + 1) * J2, :]

    for ci in range(10):
        for ph in range(7):
            m = None
            for di in range(2):
                for dj in range(2):
                    v = a2s[ci, pl.ds((2 * ph + di) * W2 + dj, 8, stride=2), :]
                    m = v if m is None else jnp.maximum(m, v)
            p3s[pl.ds(ci * 56 + ph * 8, 8), :] = m

    h1 = jnp.dot(fc1w_ref[...], p3s[...], preferred_element_type=f32)
    h1 = jnp.maximum(h1 + fc1b_ref[...], 0.0)
    out = jnp.dot(fc2w_ref[...], h1, preferred_element_type=f32)
    o_ref[...] = out + fc2b_ref[...]


@jax.jit
def kernel(x, w1, b1, w2, b2, fc1_w, fc1_b, fc2_w, fc2_b):
    B = x.shape[0]
    f32 = jnp.float32
    bf16 = jnp.bfloat16

    xp = jnp.pad(x[:, 0], ((0, 0), (1, 3), (1, 1)))
    xt = xp.reshape(B, N1).T

    oh1 = np.zeros((9, J1, K1), np.float32)
    oh2 = np.zeros((9, J2, K2), np.float32)
    for t in range(9):
        for p in range(J1):
            oh1[t, p, p + (t // 3) * W1 + (t % 3)] = 1.0
        for p in range(J2):
            oh2[t, p, p + (t // 3) * W2 + (t % 3)] = 1.0
    w1s = w1[:, 0, :10].T
    l1m = jnp.einsum('ct,tpk->cpk', w1s,
                     jnp.asarray(oh1)).reshape(240, K1)
    b1r = jnp.broadcast_to(jnp.repeat(b1[0, :10], J1)[:, None], (240, BT))

    w2s = jnp.transpose(w2[:, :10, :10], (2, 1, 0))
    l2m = jnp.einsum('cit,tpk->cpik', w2s,
                     jnp.asarray(oh2)).reshape(160, 10 * K2)
    b2r = jnp.broadcast_to(jnp.repeat(b2[0, :10], J2)[:, None], (160, BT))

    fw = fc1_w[:, :10, :].reshape(7, 7, 10, 32)
    fwp = jnp.zeros((10, 7, 8, 32), f32)
    fwp = fwp.at[:, :, :7, :].set(jnp.transpose(fw, (2, 0, 1, 3)))
    fc1wm = fwp.reshape(F1, 32).T
    fc1bb = jnp.broadcast_to(fc1_b.reshape(32, 1), (32, BT))

    fc2wm = jnp.zeros((16, 32), f32).at[:10].set(fc2_w[:, :10].T)
    fc2bb = jnp.zeros((16, BT), f32).at[:10].set(
        jnp.broadcast_to(fc2_b[0, :10, None], (10, BT)))

    grid = (B // BT,)
    out = pl.pallas_call(
        _fused_kernel,
        out_shape=jax.ShapeDtypeStruct((16, B), f32),
        grid=grid,
        in_specs=[
            pl.BlockSpec((N1, BT), lambda b: (0, b)),
            pl.BlockSpec((240, K1), lambda b: (0, 0)),
            pl.BlockSpec((240, BT), lambda b: (0, 0)),
            pl.BlockSpec((160, 10 * K2), lambda b: (0, 0)),
            pl.BlockSpec((160, BT), lambda b: (0, 0)),
            pl.BlockSpec((32, F1), lambda b: (0, 0)),
            pl.BlockSpec((32, BT), lambda b: (0, 0)),
            pl.BlockSpec((16, 32), lambda b: (0, 0)),
            pl.BlockSpec((16, BT), lambda b: (0, 0)),
        ],
        out_specs=pl.BlockSpec((16, BT), lambda b: (0, b)),
        scratch_shapes=[
            pltpu.VMEM((10, A1, BT), f32),
            pltpu.VMEM((10, N2, BT), f32),
            pltpu.VMEM((10, A2, BT), f32),
            pltpu.VMEM((F1, BT), f32),
        ],
        compiler_params=pltpu.CompilerParams(
            dimension_semantics=("parallel",)),
    )(xt, l1m, b1r, l2m, b2r, fc1wm, fc1bb, fc2wm, fc2bb)

    return out[:10].T
